# baseline (speedup 1.0000x reference)
import functools

import jax
import jax.numpy as jnp
from jax import lax
from jax.experimental import pallas as pl
from jax.experimental.pallas import tpu as pltpu
from jax.experimental.pallas import tpu_sc as plsc

K = 100
MIN_IDX = 1
NUM_ITEMS = 100000
IN = 64
HID = 64

ITEM_BLK = 2048
NUM_BLKS = 49
NPAD = ITEM_BLK * NUM_BLKS
SUB = 128
NB = NPAD // SUB

NWORKERS = 32
B = 1024
ROWS_PER_W = B // NWORKERS
CB_CAP = 128
ECAP = 512

NEG_INF = float("-inf")


def _dot(a, b, dims):
    return lax.dot_general(a, b, dims, preferred_element_type=jnp.float32)


def _score_body(items_ref, uenc_ref, scores_ref, probs_ref, bm_ref):
    i = pl.program_id(0)
    s = _dot(uenc_ref[...], items_ref[...], (((1,), (1,)), ((), ())))
    col = jax.lax.broadcasted_iota(jnp.int32, (B, ITEM_BLK), 1)
    valid = (col + i * ITEM_BLK) < NUM_ITEMS
    s = jnp.where(valid, s, NEG_INF)
    scores_ref[...] = s
    probs_ref[...] = jax.nn.sigmoid(s)
    bm_ref[...] = jnp.max(
        s.reshape(B, ITEM_BLK // SUB, SUB), axis=-1)[None, :, :]


def _splat_i32(v):
    return jnp.full((16,), v, jnp.int32)


def _sc_body(scores_hbm, bm_hbm, t_hbm, vals_hbm, idx_hbm,
             tvec_v, bmrow_v, blkl_v, blkg_v, gbuf_v, cvals_v, cidx_v, sem):
    c = lax.axis_index("c")
    s = lax.axis_index("s")
    wid = s * 2 + c
    r0 = wid * ROWS_PER_W
    il = lax.iota(jnp.int32, 16)

    def row_body(i, _):
        r = r0 + i
        pltpu.sync_copy(t_hbm.at[pl.ds(r * 16, 16)], tvec_v)
        tv = tvec_v[...]
        pltpu.sync_copy(bm_hbm.at[pl.ds(r * NB, NB)], bmrow_v)

        pad_l = _splat_i32(NB - 1)
        pad_g = _splat_i32(r * NB + NB - 1)

        def fill_blk(j, _):
            blkl_v[pl.ds(j * 16, 16)] = pad_l
            blkg_v[pl.ds(j * 16, 16)] = pad_g
            return 0

        lax.fori_loop(0, CB_CAP // 16, fill_blk, 0)

        def scan_bm(j, cb):
            bmv = bmrow_v[pl.ds(j * 16, 16)]
            m = bmv >= tv
            cnt = plsc.all_reduce_population_count(m)[0]

            def hit():
                bl = il + j * 16
                pos = jnp.minimum(cb, CB_CAP - 16) + plsc.cumsum(
                    m.astype(jnp.int32)) - 1
                pos = jnp.clip(pos, 0, CB_CAP - 1)
                plsc.store_scatter(blkl_v, [pos], bl, mask=m)
                plsc.store_scatter(blkg_v, [pos], bl + r * NB, mask=m)
                return cb + cnt

            return lax.cond(cnt > 0, hit, lambda: cb)

        lax.fori_loop(0, NB // 16, scan_bm, 0)

        copies = []
        for g in range(CB_CAP // 16):
            bvec = blkg_v[pl.ds(g * 16, 16)]
            copies.append(pltpu.async_copy(
                scores_hbm.at[bvec], gbuf_v.at[pl.ds(g * 16, 16)], sem))
        for cp in copies:
            cp.wait()

        def fill_cand(j, _):
            cvals_v[pl.ds(j * 16, 16)] = jnp.full((16,), NEG_INF, jnp.float32)
            cidx_v[pl.ds(j * 16, 16)] = _splat_i32(0)
            return 0

        lax.fori_loop(0, ECAP // 16, fill_cand, 0)

        def collect(g, ce):
            blkv = blkl_v[pl.ds(g * 16, 16)]
            for e in range(16):
                jb = g * 16 + e
                base_iv = blkv[e] * SUB + il
                for t in range(SUB // 16):
                    v = gbuf_v[jb, pl.ds(t * 16, 16)]
                    m = v >= tv
                    cnt = plsc.all_reduce_population_count(m)[0]

                    def hit(ce=ce, v=v, m=m, iv=base_iv + t * 16, cnt=cnt):
                        pos = jnp.minimum(ce, ECAP - 16) + plsc.cumsum(
                            m.astype(jnp.int32)) - 1
                        pos = jnp.clip(pos, 0, ECAP - 1)
                        plsc.store_scatter(cvals_v, [pos], v, mask=m)
                        plsc.store_scatter(cidx_v, [pos], iv, mask=m)
                        return ce + cnt

                    ce = lax.cond(cnt > 0, hit, lambda ce=ce: ce)
            return ce

        lax.fori_loop(0, CB_CAP // 16, collect, 0)

        pltpu.sync_copy(cvals_v, vals_hbm.at[pl.ds(r * ECAP, ECAP)])
        pltpu.sync_copy(cidx_v, idx_hbm.at[pl.ds(r * ECAP, ECAP)])
        return 0

    lax.fori_loop(0, ROWS_PER_W, row_body, 0)


def kernel(user_embeddings, item_embeddings, W1, b1, g1, be1, W2, b2, g2, be2):
    def _enc(x):
        h = x @ W1 + b1
        mu = jnp.mean(h, axis=-1, keepdims=True)
        var = jnp.var(h, axis=-1, keepdims=True)
        h = (h - mu) / jnp.sqrt(var + 1e-5) * g1 + be1
        h = jax.nn.relu(h)
        h = h @ W2 + b2
        mu = jnp.mean(h, axis=-1, keepdims=True)
        var = jnp.var(h, axis=-1, keepdims=True)
        return (h - mu) / jnp.sqrt(var + 1e-5) * g2 + be2

    uenc = _enc(user_embeddings)
    items = _enc(item_embeddings[MIN_IDX:])
    items_pad = jnp.concatenate(
        [items, jnp.zeros((NPAD - NUM_ITEMS, HID), jnp.float32)], axis=0)

    scores, probs, bm = pl.pallas_call(
        _score_body,
        grid=(NUM_BLKS,),
        in_specs=[
            pl.BlockSpec((ITEM_BLK, HID), lambda i: (i, 0)),
            pl.BlockSpec((B, HID), lambda i: (0, 0)),
        ],
        out_specs=[
            pl.BlockSpec((B, ITEM_BLK), lambda i: (0, i)),
            pl.BlockSpec((B, ITEM_BLK), lambda i: (0, i)),
            pl.BlockSpec((1, B, ITEM_BLK // SUB), lambda i: (i, 0, 0)),
        ],
        out_shape=[
            jax.ShapeDtypeStruct((B, NPAD), jnp.float32),
            jax.ShapeDtypeStruct((B, NPAD), jnp.float32),
            jax.ShapeDtypeStruct((NUM_BLKS, B, ITEM_BLK // SUB), jnp.float32),
        ],
        compiler_params=pltpu.CompilerParams(
            dimension_semantics=("arbitrary",),
        ),
    )(items_pad, uenc)

    bm = bm.transpose(1, 0, 2).reshape(B, NB)
    bm_vals, _ = lax.top_k(bm, K)
    t = bm_vals[:, -1]

    scores2d = scores.reshape(B * NB, SUB)
    bm_flat = bm.reshape(B * NB)

    mesh = plsc.VectorSubcoreMesh(core_axis_name="c", subcore_axis_name="s")
    sc_kernel = functools.partial(
        pl.kernel, mesh=mesh,
        compiler_params=pltpu.CompilerParams(needs_layout_passes=False),
        out_type=[
            jax.ShapeDtypeStruct((B * ECAP,), jnp.float32),
            jax.ShapeDtypeStruct((B * ECAP,), jnp.int32),
        ],
        scratch_types=[
            pltpu.VMEM((16,), jnp.float32),
            pltpu.VMEM((NB,), jnp.float32),
            pltpu.VMEM((CB_CAP,), jnp.int32),
            pltpu.VMEM((CB_CAP,), jnp.int32),
            pltpu.VMEM((CB_CAP, SUB), jnp.float32),
            pltpu.VMEM((ECAP,), jnp.float32),
            pltpu.VMEM((ECAP,), jnp.int32),
            pltpu.SemaphoreType.DMA,
        ],
    )(_sc_body)
    t16 = jnp.broadcast_to(t[:, None], (B, 16)).reshape(B * 16)
    cand_vals, cand_idx = sc_kernel(scores2d, bm_flat, t16)
    cand_vals = cand_vals.reshape(B, ECAP)
    cand_idx = cand_idx.reshape(B, ECAP)

    vals, pos = lax.top_k(cand_vals, K)
    item_idx = jnp.take_along_axis(cand_idx, pos, axis=1)
    indices = item_idx + MIN_IDX

    probs_v = probs[:, :NUM_ITEMS]
    user_loss = jnp.zeros((B,), jnp.float32)
    item_loss = jnp.zeros((NUM_ITEMS,), jnp.float32)
    add_loss = jnp.float32(0.0)
    return (indices, vals, probs_v, user_loss, item_loss, add_loss)


# speedup vs baseline: 1.2158x; 1.2158x over previous
import functools

import jax
import jax.numpy as jnp
from jax import lax
from jax.experimental import pallas as pl
from jax.experimental.pallas import tpu as pltpu
from jax.experimental.pallas import tpu_sc as plsc

K = 100
MIN_IDX = 1
NUM_ITEMS = 100000
IN = 64
HID = 64

ITEM_BLK = 2048
NUM_BLKS = 49
NPAD = ITEM_BLK * NUM_BLKS
SUB = 128
NB = NPAD // SUB

NWORKERS = 32
B = 1024
ROWS_PER_W = B // NWORKERS
CB_CAP = 128
ECAP = 512

NEG_INF = float("-inf")


def _dot(a, b, dims):
    return lax.dot_general(a, b, dims, preferred_element_type=jnp.float32)


def _score_body(items_ref, uenc_ref, scores_ref, probs_ref, bm_ref):
    i = pl.program_id(0)
    s = _dot(uenc_ref[...], items_ref[...], (((1,), (1,)), ((), ())))
    col = jax.lax.broadcasted_iota(jnp.int32, (B, ITEM_BLK), 1)
    valid = (col + i * ITEM_BLK) < NUM_ITEMS
    s = jnp.where(valid, s, NEG_INF)
    scores_ref[...] = s
    probs_ref[...] = jax.nn.sigmoid(s)
    bm_ref[...] = jnp.max(
        s.reshape(B, ITEM_BLK // SUB, SUB), axis=-1)[None, :, :]


def _splat_i32(v):
    return jnp.full((16,), v, jnp.int32)


def _sc_body(scores_hbm, bm_hbm, t_hbm, vals_hbm, idx_hbm,
             tvec_v, bmrow_v, blkl_v, blkg_v, gbuf_v, cvals_v, cidx_v, sem):
    c = lax.axis_index("c")
    s = lax.axis_index("s")
    wid = s * 2 + c
    r0 = wid * ROWS_PER_W
    il = lax.iota(jnp.int32, 16)

    def row_body(i, _):
        r = r0 + i
        pltpu.sync_copy(t_hbm.at[pl.ds(r * 16, 16)], tvec_v)
        tv = tvec_v[...]
        pltpu.sync_copy(bm_hbm.at[pl.ds(r * NB, NB)], bmrow_v)

        pad_l = _splat_i32(NB - 1)
        pad_g = _splat_i32(r * NB + NB - 1)

        def fill_blk(j, _):
            blkl_v[pl.ds(j * 16, 16)] = pad_l
            blkg_v[pl.ds(j * 16, 16)] = pad_g
            return 0

        lax.fori_loop(0, CB_CAP // 16, fill_blk, 0)

        def scan_bm(j, cb):
            bmv = bmrow_v[pl.ds(j * 16, 16)]
            m = bmv >= tv
            cnt = plsc.all_reduce_population_count(m)[0]
            bl = il + j * 16
            pos = jnp.minimum(cb, CB_CAP - 16) + plsc.cumsum(
                m.astype(jnp.int32)) - 1
            pos = jnp.clip(pos, 0, CB_CAP - 1)
            plsc.store_scatter(blkl_v, [pos], bl, mask=m)
            plsc.store_scatter(blkg_v, [pos], bl + r * NB, mask=m)
            return cb + cnt

        lax.fori_loop(0, NB // 16, scan_bm, 0)

        copies = []
        for g in range(CB_CAP // 16):
            bvec = blkg_v[pl.ds(g * 16, 16)]
            copies.append(pltpu.async_copy(
                scores_hbm.at[bvec], gbuf_v.at[pl.ds(g * 16, 16)], sem))
        for cp in copies:
            cp.wait()

        def fill_cand(j, _):
            cvals_v[pl.ds(j * 16, 16)] = jnp.full((16,), NEG_INF, jnp.float32)
            cidx_v[pl.ds(j * 16, 16)] = _splat_i32(0)
            return 0

        lax.fori_loop(0, ECAP // 16, fill_cand, 0)

        def collect(g, ce):
            blkv = blkl_v[pl.ds(g * 16, 16)]
            for e in range(16):
                jb = g * 16 + e
                base_iv = blkv[e] * SUB + il
                for t in range(SUB // 16):
                    v = gbuf_v[jb, pl.ds(t * 16, 16)]
                    m = v >= tv
                    cnt = plsc.all_reduce_population_count(m)[0]
                    iv = base_iv + t * 16
                    pos = jnp.minimum(ce, ECAP - 16) + plsc.cumsum(
                        m.astype(jnp.int32)) - 1
                    pos = jnp.clip(pos, 0, ECAP - 1)
                    plsc.store_scatter(cvals_v, [pos], v, mask=m)
                    plsc.store_scatter(cidx_v, [pos], iv, mask=m)
                    ce = ce + cnt
            return ce

        lax.fori_loop(0, CB_CAP // 16, collect, 0)

        pltpu.sync_copy(cvals_v, vals_hbm.at[pl.ds(r * ECAP, ECAP)])
        pltpu.sync_copy(cidx_v, idx_hbm.at[pl.ds(r * ECAP, ECAP)])
        return 0

    lax.fori_loop(0, ROWS_PER_W, row_body, 0)


def kernel(user_embeddings, item_embeddings, W1, b1, g1, be1, W2, b2, g2, be2):
    def _enc(x):
        h = x @ W1 + b1
        mu = jnp.mean(h, axis=-1, keepdims=True)
        var = jnp.var(h, axis=-1, keepdims=True)
        h = (h - mu) / jnp.sqrt(var + 1e-5) * g1 + be1
        h = jax.nn.relu(h)
        h = h @ W2 + b2
        mu = jnp.mean(h, axis=-1, keepdims=True)
        var = jnp.var(h, axis=-1, keepdims=True)
        return (h - mu) / jnp.sqrt(var + 1e-5) * g2 + be2

    uenc = _enc(user_embeddings)
    items = _enc(item_embeddings[MIN_IDX:])
    items_pad = jnp.concatenate(
        [items, jnp.zeros((NPAD - NUM_ITEMS, HID), jnp.float32)], axis=0)

    scores, probs, bm = pl.pallas_call(
        _score_body,
        grid=(NUM_BLKS,),
        in_specs=[
            pl.BlockSpec((ITEM_BLK, HID), lambda i: (i, 0)),
            pl.BlockSpec((B, HID), lambda i: (0, 0)),
        ],
        out_specs=[
            pl.BlockSpec((B, ITEM_BLK), lambda i: (0, i)),
            pl.BlockSpec((B, ITEM_BLK), lambda i: (0, i)),
            pl.BlockSpec((1, B, ITEM_BLK // SUB), lambda i: (i, 0, 0)),
        ],
        out_shape=[
            jax.ShapeDtypeStruct((B, NPAD), jnp.float32),
            jax.ShapeDtypeStruct((B, NPAD), jnp.float32),
            jax.ShapeDtypeStruct((NUM_BLKS, B, ITEM_BLK // SUB), jnp.float32),
        ],
        compiler_params=pltpu.CompilerParams(
            dimension_semantics=("arbitrary",),
        ),
    )(items_pad, uenc)

    bm = bm.transpose(1, 0, 2).reshape(B, NB)
    bm_vals, _ = lax.top_k(bm, K)
    t = bm_vals[:, -1]

    scores2d = scores.reshape(B * NB, SUB)
    bm_flat = bm.reshape(B * NB)

    mesh = plsc.VectorSubcoreMesh(core_axis_name="c", subcore_axis_name="s")
    sc_kernel = functools.partial(
        pl.kernel, mesh=mesh,
        compiler_params=pltpu.CompilerParams(needs_layout_passes=False),
        out_type=[
            jax.ShapeDtypeStruct((B * ECAP,), jnp.float32),
            jax.ShapeDtypeStruct((B * ECAP,), jnp.int32),
        ],
        scratch_types=[
            pltpu.VMEM((16,), jnp.float32),
            pltpu.VMEM((NB,), jnp.float32),
            pltpu.VMEM((CB_CAP,), jnp.int32),
            pltpu.VMEM((CB_CAP,), jnp.int32),
            pltpu.VMEM((CB_CAP, SUB), jnp.float32),
            pltpu.VMEM((ECAP,), jnp.float32),
            pltpu.VMEM((ECAP,), jnp.int32),
            pltpu.SemaphoreType.DMA,
        ],
    )(_sc_body)
    t16 = jnp.broadcast_to(t[:, None], (B, 16)).reshape(B * 16)
    cand_vals, cand_idx = sc_kernel(scores2d, bm_flat, t16)
    cand_vals = cand_vals.reshape(B, ECAP)
    cand_idx = cand_idx.reshape(B, ECAP)

    vals, pos = lax.top_k(cand_vals, K)
    item_idx = jnp.take_along_axis(cand_idx, pos, axis=1)
    indices = item_idx + MIN_IDX

    probs_v = probs[:, :NUM_ITEMS]
    user_loss = jnp.zeros((B,), jnp.float32)
    item_loss = jnp.zeros((NUM_ITEMS,), jnp.float32)
    add_loss = jnp.float32(0.0)
    return (indices, vals, probs_v, user_loss, item_loss, add_loss)


# speedup vs baseline: 1.2224x; 1.0054x over previous
import functools

import jax
import jax.numpy as jnp
from jax import lax
from jax.experimental import pallas as pl
from jax.experimental.pallas import tpu as pltpu
from jax.experimental.pallas import tpu_sc as plsc

K = 100
MIN_IDX = 1
NUM_ITEMS = 100000
IN = 64
HID = 64

ITEM_BLK = 2048
NUM_BLKS = 49
NPAD = ITEM_BLK * NUM_BLKS
SUB = 128
NB = NPAD // SUB

NWORKERS = 32
B = 1024
ROWS_PER_W = B // NWORKERS
CB_CAP = 128
ECAP = 512

NEG_INF = float("-inf")


def _dot(a, b, dims):
    return lax.dot_general(a, b, dims, preferred_element_type=jnp.float32)


def _score_body(items_ref, uenc_ref, scores_ref, probs_ref, bm_ref):
    i = pl.program_id(0)
    s = _dot(uenc_ref[...], items_ref[...], (((1,), (1,)), ((), ())))
    col = jax.lax.broadcasted_iota(jnp.int32, (B, ITEM_BLK), 1)
    valid = (col + i * ITEM_BLK) < NUM_ITEMS
    s = jnp.where(valid, s, NEG_INF)
    scores_ref[...] = s
    probs_ref[...] = jax.nn.sigmoid(s)
    bm_ref[...] = jnp.max(
        s.reshape(B, ITEM_BLK // SUB, SUB), axis=-1)[None, :, :]


def _splat_i32(v):
    return jnp.full((16,), v, jnp.int32)


def _sc_body(scores_hbm, bm_hbm, t_hbm, vals_hbm, idx_hbm,
             tvec_v, bmrow_v, blkl_v, blkg_v, gbuf_v, cvals_v, cidx_v, sem):
    c = lax.axis_index("c")
    s = lax.axis_index("s")
    wid = s * 2 + c
    r0 = wid * ROWS_PER_W
    il = lax.iota(jnp.int32, 16)
    pltpu.sync_copy(t_hbm.at[pl.ds(r0 * 16, ROWS_PER_W * 16)], tvec_v)

    def row_body(i, _):
        r = r0 + i
        tv = tvec_v[pl.ds(i * 16, 16)]
        pltpu.sync_copy(bm_hbm.at[pl.ds(r * NB, NB)], bmrow_v)

        pad_l = _splat_i32(NB - 1)
        pad_g = _splat_i32(r * NB + NB - 1)

        def fill_blk(j, _):
            blkl_v[pl.ds(j * 16, 16)] = pad_l
            blkg_v[pl.ds(j * 16, 16)] = pad_g
            return 0

        lax.fori_loop(0, CB_CAP // 16, fill_blk, 0)

        def scan_bm(j, cb):
            bmv = bmrow_v[pl.ds(j * 16, 16)]
            m = bmv >= tv
            cnt = plsc.all_reduce_population_count(m)[0]
            bl = il + j * 16
            pos = jnp.minimum(cb, CB_CAP - 16) + plsc.cumsum(
                m.astype(jnp.int32)) - 1
            pos = jnp.clip(pos, 0, CB_CAP - 1)
            plsc.store_scatter(blkl_v, [pos], bl, mask=m)
            plsc.store_scatter(blkg_v, [pos], bl + r * NB, mask=m)
            return cb + cnt

        lax.fori_loop(0, NB // 16, scan_bm, 0)

        copies = []
        for g in range(CB_CAP // 16):
            bvec = blkg_v[pl.ds(g * 16, 16)]
            copies.append(pltpu.async_copy(
                scores_hbm.at[bvec], gbuf_v.at[pl.ds(g * 16, 16)], sem))
        for cp in copies:
            cp.wait()

        def fill_cand(j, _):
            cvals_v[pl.ds(j * 16, 16)] = jnp.full((16,), NEG_INF, jnp.float32)
            cidx_v[pl.ds(j * 16, 16)] = _splat_i32(0)
            return 0

        lax.fori_loop(0, ECAP // 16, fill_cand, 0)

        def collect(g, ce):
            blkv = blkl_v[pl.ds(g * 16, 16)]
            for e in range(16):
                jb = g * 16 + e
                base_iv = blkv[e] * SUB + il
                for t in range(SUB // 16):
                    v = gbuf_v[jb, pl.ds(t * 16, 16)]
                    m = v >= tv
                    cnt = plsc.all_reduce_population_count(m)[0]
                    iv = base_iv + t * 16
                    pos = jnp.minimum(ce, ECAP - 16) + plsc.cumsum(
                        m.astype(jnp.int32)) - 1
                    pos = jnp.clip(pos, 0, ECAP - 1)
                    plsc.store_scatter(cvals_v, [pos], v, mask=m)
                    plsc.store_scatter(cidx_v, [pos], iv, mask=m)
                    ce = ce + cnt
            return ce

        lax.fori_loop(0, CB_CAP // 16, collect, 0)

        pltpu.sync_copy(cvals_v, vals_hbm.at[pl.ds(r * ECAP, ECAP)])
        pltpu.sync_copy(cidx_v, idx_hbm.at[pl.ds(r * ECAP, ECAP)])
        return 0

    lax.fori_loop(0, ROWS_PER_W, row_body, 0)


def kernel(user_embeddings, item_embeddings, W1, b1, g1, be1, W2, b2, g2, be2):
    def _enc(x):
        h = x @ W1 + b1
        mu = jnp.mean(h, axis=-1, keepdims=True)
        var = jnp.var(h, axis=-1, keepdims=True)
        h = (h - mu) / jnp.sqrt(var + 1e-5) * g1 + be1
        h = jax.nn.relu(h)
        h = h @ W2 + b2
        mu = jnp.mean(h, axis=-1, keepdims=True)
        var = jnp.var(h, axis=-1, keepdims=True)
        return (h - mu) / jnp.sqrt(var + 1e-5) * g2 + be2

    uenc = _enc(user_embeddings)
    items = _enc(item_embeddings[MIN_IDX:])
    items_pad = jnp.concatenate(
        [items, jnp.zeros((NPAD - NUM_ITEMS, HID), jnp.float32)], axis=0)

    scores, probs, bm = pl.pallas_call(
        _score_body,
        grid=(NUM_BLKS,),
        in_specs=[
            pl.BlockSpec((ITEM_BLK, HID), lambda i: (i, 0)),
            pl.BlockSpec((B, HID), lambda i: (0, 0)),
        ],
        out_specs=[
            pl.BlockSpec((B, ITEM_BLK), lambda i: (0, i)),
            pl.BlockSpec((B, ITEM_BLK), lambda i: (0, i)),
            pl.BlockSpec((1, B, ITEM_BLK // SUB), lambda i: (i, 0, 0)),
        ],
        out_shape=[
            jax.ShapeDtypeStruct((B, NPAD), jnp.float32),
            jax.ShapeDtypeStruct((B, NPAD), jnp.float32),
            jax.ShapeDtypeStruct((NUM_BLKS, B, ITEM_BLK // SUB), jnp.float32),
        ],
        compiler_params=pltpu.CompilerParams(
            dimension_semantics=("arbitrary",),
        ),
    )(items_pad, uenc)

    bm = bm.transpose(1, 0, 2).reshape(B, NB)
    bm_vals, _ = lax.top_k(bm, K)
    t = bm_vals[:, -1]

    scores2d = scores.reshape(B * NB, SUB)
    bm_flat = bm.reshape(B * NB)

    mesh = plsc.VectorSubcoreMesh(core_axis_name="c", subcore_axis_name="s")
    sc_kernel = functools.partial(
        pl.kernel, mesh=mesh,
        compiler_params=pltpu.CompilerParams(needs_layout_passes=False),
        out_type=[
            jax.ShapeDtypeStruct((B * ECAP,), jnp.float32),
            jax.ShapeDtypeStruct((B * ECAP,), jnp.int32),
        ],
        scratch_types=[
            pltpu.VMEM((ROWS_PER_W * 16,), jnp.float32),
            pltpu.VMEM((NB,), jnp.float32),
            pltpu.VMEM((CB_CAP,), jnp.int32),
            pltpu.VMEM((CB_CAP,), jnp.int32),
            pltpu.VMEM((CB_CAP, SUB), jnp.float32),
            pltpu.VMEM((ECAP,), jnp.float32),
            pltpu.VMEM((ECAP,), jnp.int32),
            pltpu.SemaphoreType.DMA,
        ],
    )(_sc_body)
    t16 = jnp.broadcast_to(t[:, None], (B, 16)).reshape(B * 16)
    cand_vals, cand_idx = sc_kernel(scores2d, bm_flat, t16)
    cand_vals = cand_vals.reshape(B, ECAP)
    cand_idx = cand_idx.reshape(B, ECAP)

    vals, pos = lax.top_k(cand_vals, K)
    item_idx = jnp.take_along_axis(cand_idx, pos, axis=1)
    indices = item_idx + MIN_IDX

    probs_v = probs[:, :NUM_ITEMS]
    user_loss = jnp.zeros((B,), jnp.float32)
    item_loss = jnp.zeros((NUM_ITEMS,), jnp.float32)
    add_loss = jnp.float32(0.0)
    return (indices, vals, probs_v, user_loss, item_loss, add_loss)
